# baseline (speedup 1.0000x reference)
import functools

import jax
import jax.numpy as jnp
from jax import lax
from jax.experimental import pallas as pl
from jax.experimental.pallas import tpu as pltpu
from jax.experimental.pallas import tpu_sc as plsc

VOCAB = 1000
HIDDEN = 128
BATCH_TOKENS = 1024 * 50

NC = 2
NS = 16
NW = NC * NS

B_PER_W = BATCH_TOKENS // NW
CHUNK = 80
N_CHUNKS = B_PER_W // CHUNK

assert B_PER_W * NW == BATCH_TOKENS
assert N_CHUNKS * CHUNK == B_PER_W
assert CHUNK % 8 == 0


def _table_body(embed_ref, w_ref, b_ref, p_ref):
    p_ref[...] = (
        lax.dot_general(
            embed_ref[...],
            w_ref[...],
            (((1,), (1,)), ((), ())),
            preferred_element_type=jnp.float32,
            precision=lax.Precision.HIGHEST,
        )
        + b_ref[...]
    )


def _compute_table(embed, W, b):
    return pl.pallas_call(
        _table_body,
        out_shape=jax.ShapeDtypeStruct((VOCAB, VOCAB), jnp.float32),
    )(embed, W, b.reshape(1, VOCAB))


_MESH = plsc.VectorSubcoreMesh(
    core_axis_name="c", subcore_axis_name="s", num_cores=NC, num_subcores=NS
)


@functools.partial(
    pl.kernel,
    out_type=jax.ShapeDtypeStruct((BATCH_TOKENS, VOCAB), jnp.float32),
    mesh=_MESH,
    scratch_types=[
        pltpu.VMEM((CHUNK,), jnp.int32),
        pltpu.VMEM((CHUNK, VOCAB), jnp.float32),
        pltpu.SemaphoreType.DMA,
    ],
    compiler_params=pltpu.CompilerParams(use_tc_tiling_on_sc=False),
)
def _gather_rows(table_hbm, idx_hbm, out_hbm, idx_v, rows_v, sem):
    wid = lax.axis_index("s") * NC + lax.axis_index("c")
    base = wid * B_PER_W

    def step(c, carry):
        off = pl.multiple_of(base + c * CHUNK, 8)
        pltpu.sync_copy(idx_hbm.at[pl.ds(off, CHUNK)], idx_v)
        pltpu.async_copy(table_hbm.at[idx_v], rows_v, sem).wait()
        pltpu.sync_copy(rows_v, out_hbm.at[pl.ds(off, CHUNK)])
        return carry

    lax.fori_loop(0, N_CHUNKS, step, 0)


def kernel(input_ids, embed, W, b):
    batch, seq = input_ids.shape
    table = _compute_table(embed, W, b)
    ids = input_ids.reshape(-1).astype(jnp.int32)
    out = _gather_rows(table, ids)
    return out.reshape(batch, seq, VOCAB)


# speedup vs baseline: 1.0187x; 1.0187x over previous
import functools

import jax
import jax.numpy as jnp
from jax import lax
from jax.experimental import pallas as pl
from jax.experimental.pallas import tpu as pltpu
from jax.experimental.pallas import tpu_sc as plsc

VOCAB = 1000
HIDDEN = 128
BATCH_TOKENS = 1024 * 50

NC = 2
NS = 16
NW = NC * NS

B_PER_W = BATCH_TOKENS // NW
CHUNK = 40
N_CHUNKS = B_PER_W // CHUNK

assert B_PER_W * NW == BATCH_TOKENS
assert N_CHUNKS * CHUNK == B_PER_W
assert N_CHUNKS % 2 == 0
assert CHUNK % 8 == 0


def _table_body(embed_ref, w_ref, b_ref, p_ref):
    p_ref[...] = (
        lax.dot_general(
            embed_ref[...],
            w_ref[...],
            (((1,), (1,)), ((), ())),
            preferred_element_type=jnp.float32,
            precision=lax.Precision.HIGHEST,
        )
        + b_ref[...]
    )


def _compute_table(embed, W, b):
    return pl.pallas_call(
        _table_body,
        out_shape=jax.ShapeDtypeStruct((VOCAB, VOCAB), jnp.float32),
    )(embed, W, b.reshape(1, VOCAB))


_MESH = plsc.VectorSubcoreMesh(
    core_axis_name="c", subcore_axis_name="s", num_cores=NC, num_subcores=NS
)


@functools.partial(
    pl.kernel,
    out_type=jax.ShapeDtypeStruct((BATCH_TOKENS, VOCAB), jnp.float32),
    mesh=_MESH,
    scratch_types=[
        pltpu.VMEM((B_PER_W,), jnp.int32),
        pltpu.VMEM((CHUNK, VOCAB), jnp.float32),
        pltpu.VMEM((CHUNK, VOCAB), jnp.float32),
        pltpu.SemaphoreType.DMA,
        pltpu.SemaphoreType.DMA,
        pltpu.SemaphoreType.DMA,
        pltpu.SemaphoreType.DMA,
    ],
    compiler_params=pltpu.CompilerParams(use_tc_tiling_on_sc=False),
)
def _gather_rows(
    table_hbm, idx_hbm, out_hbm,
    idx_v, rows0, rows1, sem_g0, sem_g1, sem_w0, sem_w1,
):
    wid = lax.axis_index("s") * NC + lax.axis_index("c")
    base = wid * B_PER_W

    rows = (rows0, rows1)
    sem_g = (sem_g0, sem_g1)
    sem_w = (sem_w0, sem_w1)

    def idx_slice(c):
        return idx_v.at[pl.ds(pl.multiple_of(c * CHUNK, 8), CHUNK)]

    def out_slice(c):
        return out_hbm.at[pl.ds(base + c * CHUNK, CHUNK)]

    def start_gather(c, b):
        pltpu.async_copy(table_hbm.at[idx_slice(c)], rows[b], sem_g[b])

    def wait_gather(b):
        pltpu.make_async_copy(table_hbm.at[idx_slice(0)], rows[b], sem_g[b]).wait()

    def start_write(c, b):
        pltpu.async_copy(rows[b], out_slice(c), sem_w[b])

    def wait_write(b):
        pltpu.make_async_copy(rows[b], out_slice(0), sem_w[b]).wait()

    pltpu.sync_copy(idx_hbm.at[pl.ds(base, B_PER_W)], idx_v)

    start_gather(0, 0)
    wait_gather(0)
    start_write(0, 0)
    start_gather(1, 1)

    def pair(t, carry):
        k0 = 2 * t
        wait_write(0)
        start_gather(k0, 0)
        wait_gather(1)
        start_write(k0 - 1, 1)
        wait_write(1)
        start_gather(k0 + 1, 1)
        wait_gather(0)
        start_write(k0, 0)
        return carry

    lax.fori_loop(1, N_CHUNKS // 2, pair, 0)

    wait_gather(1)
    start_write(N_CHUNKS - 1, 1)
    wait_write(0)
    wait_write(1)


def kernel(input_ids, embed, W, b):
    batch, seq = input_ids.shape
    table = _compute_table(embed, W, b)
    ids = input_ids.reshape(-1).astype(jnp.int32)
    out = _gather_rows(table, ids)
    return out.reshape(batch, seq, VOCAB)


# speedup vs baseline: 1.1273x; 1.1066x over previous
import functools

import jax
import jax.numpy as jnp
from jax import lax
from jax.experimental import pallas as pl
from jax.experimental.pallas import tpu as pltpu
from jax.experimental.pallas import tpu_sc as plsc

VOCAB = 1000
HIDDEN = 128
BATCH_TOKENS = 1024 * 50

NC = 2
NS = 16
NW = NC * NS

B_PER_W = BATCH_TOKENS // NW
CHUNK = 32
N_CHUNKS = B_PER_W // CHUNK

assert B_PER_W * NW == BATCH_TOKENS
assert N_CHUNKS * CHUNK == B_PER_W
assert N_CHUNKS % 2 == 0
assert CHUNK % 8 == 0


def _table_body(embed_ref, w_ref, b_ref, p_ref):
    p_ref[...] = (
        lax.dot_general(
            embed_ref[...],
            w_ref[...],
            (((1,), (1,)), ((), ())),
            preferred_element_type=jnp.float32,
            precision=lax.Precision.HIGHEST,
        )
        + b_ref[...]
    )


def _compute_table(embed, W, b):
    return pl.pallas_call(
        _table_body,
        out_shape=jax.ShapeDtypeStruct((VOCAB, VOCAB), jnp.float32),
    )(embed, W, b.reshape(1, VOCAB))


_MESH = plsc.VectorSubcoreMesh(
    core_axis_name="c", subcore_axis_name="s", num_cores=NC, num_subcores=NS
)


@functools.partial(
    pl.kernel,
    out_type=jax.ShapeDtypeStruct((BATCH_TOKENS, VOCAB), jnp.float32),
    mesh=_MESH,
    scratch_types=[
        pltpu.VMEM_SHARED((VOCAB, VOCAB), jnp.float32),
        pltpu.VMEM((B_PER_W,), jnp.int32),
        pltpu.VMEM((CHUNK, VOCAB), jnp.float32),
        pltpu.VMEM((CHUNK, VOCAB), jnp.float32),
        pltpu.SemaphoreType.DMA,
        pltpu.SemaphoreType.DMA,
        pltpu.SemaphoreType.DMA,
        pltpu.SemaphoreType.DMA,
    ],
    compiler_params=pltpu.CompilerParams(use_tc_tiling_on_sc=False),
)
def _gather_rows(
    table_hbm, idx_hbm, out_hbm,
    tbl_sh, idx_v, rows0, rows1, sem_g0, sem_g1, sem_w0, sem_w1,
):
    cid = lax.axis_index("c")
    sid = lax.axis_index("s")
    wid = sid * NC + cid
    base = wid * B_PER_W

    @pl.when(sid < 8)
    def _():
        pltpu.sync_copy(
            table_hbm.at[pl.ds(sid * (VOCAB // 8), VOCAB // 8)],
            tbl_sh.at[pl.ds(sid * (VOCAB // 8), VOCAB // 8)],
        )

    plsc.subcore_barrier()

    rows = (rows0, rows1)
    sem_g = (sem_g0, sem_g1)
    sem_w = (sem_w0, sem_w1)

    def idx_slice(c):
        return idx_v.at[pl.ds(pl.multiple_of(c * CHUNK, 8), CHUNK)]

    def out_slice(c):
        return out_hbm.at[pl.ds(base + c * CHUNK, CHUNK)]

    def start_gather(c, b):
        pltpu.async_copy(tbl_sh.at[idx_slice(c)], rows[b], sem_g[b])

    def wait_gather(b):
        pltpu.make_async_copy(tbl_sh.at[idx_slice(0)], rows[b], sem_g[b]).wait()

    def start_write(c, b):
        pltpu.async_copy(rows[b], out_slice(c), sem_w[b])

    def wait_write(b):
        pltpu.make_async_copy(rows[b], out_slice(0), sem_w[b]).wait()

    pltpu.sync_copy(idx_hbm.at[pl.ds(base, B_PER_W)], idx_v)

    start_gather(0, 0)
    wait_gather(0)
    start_write(0, 0)
    start_gather(1, 1)

    def pair(t, carry):
        k0 = 2 * t
        wait_write(0)
        start_gather(k0, 0)
        wait_gather(1)
        start_write(k0 - 1, 1)
        wait_write(1)
        start_gather(k0 + 1, 1)
        wait_gather(0)
        start_write(k0, 0)
        return carry

    lax.fori_loop(1, N_CHUNKS // 2, pair, 0)

    wait_gather(1)
    start_write(N_CHUNKS - 1, 1)
    wait_write(0)
    wait_write(1)


def kernel(input_ids, embed, W, b):
    batch, seq = input_ids.shape
    table = _compute_table(embed, W, b)
    ids = input_ids.reshape(-1).astype(jnp.int32)
    out = _gather_rows(table, ids)
    return out.reshape(batch, seq, VOCAB)


# speedup vs baseline: 1.3303x; 1.1801x over previous
import functools

import jax
import jax.numpy as jnp
from jax import lax
from jax.experimental import pallas as pl
from jax.experimental.pallas import tpu as pltpu
from jax.experimental.pallas import tpu_sc as plsc

VOCAB = 1000
HIDDEN = 128
BATCH = 1024
SEQ = 50
TPAD = 1024
NBLK = VOCAB // 8

NC = 2
NS = 16
NW = NC * NS

VBLK = 4
VROWS = 8 * VBLK
LAST_START = NBLK - VBLK
NBT = BATCH // 128
NGRP = BATCH // 16


def _table_body(w_ref, e_ref, b_ref, out_ref):
    m = (
        lax.dot_general(
            w_ref[...],
            e_ref[...],
            (((1,), (1,)), ((), ())),
            preferred_element_type=jnp.float32,
            precision=lax.Precision.HIGHEST,
        )
        + b_ref[...]
    )
    for j in range(8):
        out_ref[:, j, :] = m[:, 128 * j : 128 * (j + 1)]


def _compute_table(embed, W, b):
    embed_pad = jnp.pad(embed, ((0, TPAD - VOCAB), (0, 0)))
    out3 = pl.pallas_call(
        _table_body,
        out_shape=jax.ShapeDtypeStruct((VOCAB, 8, 128), jnp.float32),
    )(W, embed_pad, b.reshape(VOCAB, 1))
    return out3.reshape(VOCAB, TPAD)


_MESH = plsc.VectorSubcoreMesh(
    core_axis_name="c", subcore_axis_name="s", num_cores=NC, num_subcores=NS
)


@functools.partial(
    pl.kernel,
    out_type=jax.ShapeDtypeStruct((SEQ, NBLK, 8, 8, 128), jnp.float32),
    mesh=_MESH,
    scratch_types=[
        pltpu.VMEM((VROWS, TPAD), jnp.float32),
        pltpu.VMEM((BATCH,), jnp.int32),
        pltpu.VMEM((VBLK, 8, 8, 128), jnp.float32),
        pltpu.VMEM((VBLK, 8, 8, 128), jnp.float32),
        pltpu.SemaphoreType.DMA,
        pltpu.SemaphoreType.DMA,
    ],
    compiler_params=pltpu.CompilerParams(
        use_tc_tiling_on_sc=False, needs_layout_passes=False
    ),
)
def _lookup(table_hbm, ids_hbm, out_hbm, tbl, idx_v, pan0, pan1, sem0, sem1):
    wid = lax.axis_index("s") * NC + lax.axis_index("c")
    bs = jnp.minimum(VBLK * wid, LAST_START)

    pltpu.sync_copy(table_hbm.at[pl.ds(8 * bs, VROWS)], tbl)

    pans = (pan0, pan1)
    sems = (sem0, sem1)

    def compute(s, p):
        pltpu.sync_copy(ids_hbm.at[s], idx_v)

        def grp(g, carry):
            bt = g // 8
            gr = g - 8 * bt
            tv = idx_v[pl.ds(g * 16, 16)]
            for vb in range(VBLK):
                for vs in range(8):
                    vvec = jnp.full((16,), 8 * vb + vs, jnp.int32)
                    val = plsc.load_gather(tbl, [vvec, tv])
                    pans[p][vb, bt, vs, pl.ds(gr * 16, 16)] = val
            return carry

        lax.fori_loop(0, NGRP, grp, 0)

    def start_write(s, p):
        pltpu.async_copy(pans[p], out_hbm.at[s, pl.ds(bs, VBLK)], sems[p])

    def wait_write(p):
        pltpu.make_async_copy(pans[p], out_hbm.at[0, pl.ds(bs, VBLK)], sems[p]).wait()

    compute(0, 0)
    start_write(0, 0)
    compute(1, 1)
    start_write(1, 1)

    def pair(t, carry):
        s0 = 2 * t
        wait_write(0)
        compute(s0, 0)
        start_write(s0, 0)
        wait_write(1)
        compute(s0 + 1, 1)
        start_write(s0 + 1, 1)
        return carry

    lax.fori_loop(1, SEQ // 2, pair, 0)
    wait_write(0)
    wait_write(1)


def kernel(input_ids, embed, W, b):
    table = _compute_table(embed, W, b)
    ids_t = input_ids.T.astype(jnp.int32)
    out5 = _lookup(table, ids_t)
    x = out5.transpose(2, 4, 0, 1, 3)
    return x.reshape(BATCH, SEQ, VOCAB)


# speedup vs baseline: 2.8516x; 2.1436x over previous
import functools

import jax
import jax.numpy as jnp
from jax import lax
from jax.experimental import pallas as pl
from jax.experimental.pallas import tpu as pltpu
from jax.experimental.pallas import tpu_sc as plsc

VOCAB = 1000
HIDDEN = 128
BATCH = 1024
SEQ = 50
TPAD = 1024
NBLK = VOCAB // 8

NC = 2
NS = 16
NW = NC * NS

VBLK = 4
VROWS = 8 * VBLK
LAST_START = NBLK - VBLK
NBT = BATCH // 128
NGRP = BATCH // 16


def _table_body(w_ref, e_ref, b_ref, out_ref):
    m = (
        lax.dot_general(
            w_ref[...],
            e_ref[...],
            (((1,), (1,)), ((), ())),
            preferred_element_type=jnp.float32,
            precision=lax.Precision.HIGHEST,
        )
        + b_ref[...]
    )
    for j in range(8):
        out_ref[:, j, :] = m[:, 128 * j : 128 * (j + 1)]


def _compute_table(embed, W, b):
    embed_pad = jnp.pad(embed, ((0, TPAD - VOCAB), (0, 0)))
    out3 = pl.pallas_call(
        _table_body,
        out_shape=jax.ShapeDtypeStruct((VOCAB, 8, 128), jnp.float32),
    )(W, embed_pad, b.reshape(VOCAB, 1))
    return out3.reshape(VOCAB, TPAD)


_MESH = plsc.VectorSubcoreMesh(
    core_axis_name="c", subcore_axis_name="s", num_cores=NC, num_subcores=NS
)


@functools.partial(
    pl.kernel,
    out_type=jax.ShapeDtypeStruct((SEQ, NBLK, 8, 8, 128), jnp.float32),
    mesh=_MESH,
    scratch_types=[
        pltpu.VMEM((VROWS, TPAD), jnp.float32),
        pltpu.VMEM((BATCH,), jnp.int32),
        pltpu.VMEM((VBLK, 8, 8, 128), jnp.float32),
        pltpu.VMEM((VBLK, 8, 8, 128), jnp.float32),
        pltpu.SemaphoreType.DMA,
        pltpu.SemaphoreType.DMA,
    ],
    compiler_params=pltpu.CompilerParams(
        use_tc_tiling_on_sc=False, needs_layout_passes=False
    ),
)
def _lookup(table_hbm, ids_hbm, out_hbm, tbl, idx_v, pan0, pan1, sem0, sem1):
    wid = lax.axis_index("s") * NC + lax.axis_index("c")
    bs = jnp.minimum(VBLK * wid, LAST_START)

    pltpu.sync_copy(table_hbm.at[pl.ds(8 * bs, VROWS)], tbl)

    pans = (pan0, pan1)
    sems = (sem0, sem1)

    def compute(s, p):
        pltpu.sync_copy(ids_hbm.at[s], idx_v)

        @plsc.parallel_loop(0, NGRP, 1, unroll=2)
        def _(g):
            bt = g // 8
            gr = g - 8 * bt
            tv = idx_v[pl.ds(g * 16, 16)]
            for vb in range(VBLK):
                for vs in range(8):
                    vvec = jnp.full((16,), 8 * vb + vs, jnp.int32)
                    val = plsc.load_gather(tbl, [vvec, tv])
                    pans[p][vb, bt, vs, pl.ds(gr * 16, 16)] = val

    def start_write(s, p):
        pltpu.async_copy(pans[p], out_hbm.at[s, pl.ds(bs, VBLK)], sems[p])

    def wait_write(p):
        pltpu.make_async_copy(pans[p], out_hbm.at[0, pl.ds(bs, VBLK)], sems[p]).wait()

    compute(0, 0)
    start_write(0, 0)
    compute(1, 1)
    start_write(1, 1)

    def pair(t, carry):
        s0 = 2 * t
        wait_write(0)
        compute(s0, 0)
        start_write(s0, 0)
        wait_write(1)
        compute(s0 + 1, 1)
        start_write(s0 + 1, 1)
        return carry

    lax.fori_loop(1, SEQ // 2, pair, 0)
    wait_write(0)
    wait_write(1)


def kernel(input_ids, embed, W, b):
    table = _compute_table(embed, W, b)
    ids_t = input_ids.T.astype(jnp.int32)
    out5 = _lookup(table, ids_t)
    x = out5.transpose(2, 4, 0, 1, 3)
    return x.reshape(BATCH, SEQ, VOCAB)
